# baseline (speedup 1.0000x reference)
import functools

import jax
import jax.numpy as jnp
from jax import lax
from jax.experimental import pallas as pl
from jax.experimental.pallas import tpu as pltpu
from jax.experimental.pallas import tpu_sc as plsc

DIM = 64
_info = plsc.get_sparse_core_info()
NC, NS = _info.num_cores, _info.num_subcores
NW = NC * NS

BLK = 128
SEQ_LEN = 200
BT = 4096 // BLK
N_BLOCKS = SEQ_LEN * BT
BLOCKS_PER_W = N_BLOCKS // NW
CHUNK = 2 * BLK
CHUNKS_PER_W = BLOCKS_PER_W // 2


def _body(idx_hbm, tab_hbm, out_hbm,
          idx_all, rows0, rows1, blk0, blk1,
          sem_g, sem_s):
  wid = lax.axis_index("s") * NC + lax.axis_index("c")
  base_blk = wid * BLOCKS_PER_W
  base_idx = base_blk * BLK

  pltpu.sync_copy(idx_hbm.at[pl.ds(base_idx, BLOCKS_PER_W * BLK)], idx_all)

  iota16 = lax.iota(jnp.int32, 16)
  jvecs = [jnp.full((16,), j0, jnp.int32) + iota16 for j0 in range(0, CHUNK, 16)]

  rows = (rows0, rows1)
  blks = (blk0, blk1)

  def start_gather(c, slot):
    pltpu.make_async_copy(tab_hbm.at[idx_all.at[pl.ds(c * CHUNK, CHUNK)]],
                          rows[slot], sem_g.at[slot]).start()

  def wait_gather(slot):
    pltpu.make_async_copy(tab_hbm.at[idx_all.at[pl.ds(0, CHUNK)]],
                          rows[slot], sem_g.at[slot]).wait()

  def transpose(rows_ref, blk_ref):
    @plsc.parallel_loop(0, DIM, step=8)
    def dstep(d0):
      base = d0 * (2 * BLK)
      for sub in range(2):
        for d1 in range(8):
          col = jnp.full((16,), d1, jnp.int32) + d0
          gs = [plsc.load_gather(rows_ref, [jvecs[sub * 8 + jg], col])
                for jg in range(8)]
          off = base + sub * (8 * BLK) + d1 * BLK
          for jg in range(8):
            blk_ref[pl.ds(off + jg * 16, 16)] = gs[jg]

  def start_store(c, blk_ref, slot):
    g = base_blk + 2 * c
    s = lax.div(g, BT)
    bt = lax.rem(g, BT)
    for d8 in range(8):
      off = ((s * 8 + d8) * BT + bt) * (8 * BLK)
      pltpu.make_async_copy(blk_ref.at[pl.ds(d8 * 2 * 8 * BLK, 2 * 8 * BLK)],
                            out_hbm.at[pl.ds(off, 2 * 8 * BLK)],
                            sem_s.at[slot]).start()

  def wait_store(blk_ref, slot):
    for d8 in range(8):
      pltpu.make_async_copy(blk_ref.at[pl.ds(d8 * 2 * 8 * BLK, 2 * 8 * BLK)],
                            out_hbm.at[pl.ds(0, 2 * 8 * BLK)],
                            sem_s.at[slot]).wait()

  start_gather(0, 0)

  def pair_body(p, _):
    c0 = 2 * p
    for q in range(2):
      c = c0 + q
      start_gather(c + 1, 1 - q)
      wait_gather(q)

      @pl.when(p > 0)
      def _():
        wait_store(blks[q], q)

      transpose(rows[q], blks[q])
      start_store(c, blks[q], q)
    return 0

  lax.fori_loop(0, CHUNKS_PER_W // 2 - 1, pair_body, 0, unroll=False)
  for q in range(2):
    c = CHUNKS_PER_W - 2 + q
    if q == 0:
      start_gather(c + 1, 1)
    wait_gather(q)
    wait_store(blks[q], q)
    transpose(rows[q], blks[q])
    start_store(c, blks[q], q)
  for q in range(2):
    wait_store(blks[q], q)


def kernel(indices, table):
  batch, seq = indices.shape
  n = batch * seq
  vocab = table.shape[0]
  v8 = (vocab + 7) // 8 * 8

  idx_t = (indices.T.reshape(n) * 2).astype(jnp.int32)
  tabv = jnp.pad(table, ((0, v8 - vocab), (0, 2 * DIM - table.shape[1])))
  tabv = tabv.reshape(2 * v8, DIM)

  mesh = plsc.VectorSubcoreMesh(core_axis_name="c", subcore_axis_name="s")
  k = functools.partial(
      pl.kernel,
      mesh=mesh,
      out_type=jax.ShapeDtypeStruct((n * DIM,), jnp.float32),
      scratch_types=[
          pltpu.VMEM((BLOCKS_PER_W * BLK,), jnp.int32),
          pltpu.VMEM((CHUNK, DIM), jnp.float32),
          pltpu.VMEM((CHUNK, DIM), jnp.float32),
          pltpu.VMEM((2 * DIM * BLK,), jnp.float32),
          pltpu.VMEM((2 * DIM * BLK,), jnp.float32),
          pltpu.SemaphoreType.DMA((2,)),
          pltpu.SemaphoreType.DMA((2,)),
      ],
      compiler_params=pltpu.CompilerParams(
          use_tc_tiling_on_sc=False, needs_layout_passes=False),
  )(_body)

  out_flat = k(idx_t, tabv)
  out5 = out_flat.reshape(seq, DIM // 8, batch // BLK, 8, BLK)
  return out5.transpose(2, 4, 0, 1, 3).reshape(batch, seq, DIM)


# speedup vs baseline: 2.1439x; 2.1439x over previous
import functools

import jax
import jax.numpy as jnp
from jax import lax
from jax.experimental import pallas as pl
from jax.experimental.pallas import tpu as pltpu
from jax.experimental.pallas import tpu_sc as plsc

DIM = 64
_info = plsc.get_sparse_core_info()
NC, NS = _info.num_cores, _info.num_subcores
NW = NC * NS

BLK = 128
SEQ_LEN = 200
BT = 4096 // BLK
N_BLOCKS = SEQ_LEN * BT
BLOCKS_PER_W = N_BLOCKS // NW
CHUNK = 2 * BLK
CHUNKS_PER_W = BLOCKS_PER_W // 2


def _body(idx_hbm, tab_hbm, out_hbm,
          idx_all, rows0, rows1, blk0, blk1,
          sem_g, sem_s):
  wid = lax.axis_index("s") * NC + lax.axis_index("c")
  base_blk = wid * BLOCKS_PER_W
  base_idx = base_blk * BLK

  pltpu.sync_copy(idx_hbm.at[pl.ds(base_idx, BLOCKS_PER_W * BLK)], idx_all)

  iota16 = lax.iota(jnp.int32, 16)
  jvecs = [jnp.full((16,), j0, jnp.int32) + iota16 for j0 in range(0, CHUNK, 16)]

  rows = (rows0, rows1)
  blks = (blk0, blk1)

  def start_gather(c, slot):
    pltpu.make_async_copy(tab_hbm.at[idx_all.at[pl.ds(c * CHUNK, CHUNK)]],
                          rows[slot], sem_g.at[slot]).start()

  def wait_gather(slot):
    pltpu.make_async_copy(tab_hbm.at[idx_all.at[pl.ds(0, CHUNK)]],
                          rows[slot], sem_g.at[slot]).wait()

  jconsts = [jvecs[sub * 8 + jg] + (sub * 7 * BLK)
             for sub in range(2) for jg in range(8)]

  def transpose(rows_ref, blk_ref):
    @plsc.parallel_loop(0, DIM)
    def dstep(d):
      t = jnp.bitwise_and(iota16 + d, DIM - 1)
      tpart = jnp.left_shift(jnp.right_shift(t, 3), 11) + jnp.left_shift(
          jnp.bitwise_and(t, 7), 7)
      for g in range(16):
        addr = tpart + jconsts[g]
        v = plsc.load_gather(rows_ref, [jvecs[g], t])
        plsc.store_scatter(blk_ref, [addr], v)

  def start_store(c, blk_ref, slot):
    g = base_blk + 2 * c
    s = lax.div(g, BT)
    bt = lax.rem(g, BT)
    for d8 in range(8):
      off = ((s * 8 + d8) * BT + bt) * (8 * BLK)
      pltpu.make_async_copy(blk_ref.at[pl.ds(d8 * 2 * 8 * BLK, 2 * 8 * BLK)],
                            out_hbm.at[pl.ds(off, 2 * 8 * BLK)],
                            sem_s.at[slot]).start()

  def wait_store(blk_ref, slot):
    for d8 in range(8):
      pltpu.make_async_copy(blk_ref.at[pl.ds(d8 * 2 * 8 * BLK, 2 * 8 * BLK)],
                            out_hbm.at[pl.ds(0, 2 * 8 * BLK)],
                            sem_s.at[slot]).wait()

  start_gather(0, 0)

  def pair_body(p, _):
    c0 = 2 * p
    for q in range(2):
      c = c0 + q
      start_gather(c + 1, 1 - q)
      wait_gather(q)

      @pl.when(p > 0)
      def _():
        wait_store(blks[q], q)

      transpose(rows[q], blks[q])
      start_store(c, blks[q], q)
    return 0

  lax.fori_loop(0, CHUNKS_PER_W // 2 - 1, pair_body, 0, unroll=False)
  for q in range(2):
    c = CHUNKS_PER_W - 2 + q
    if q == 0:
      start_gather(c + 1, 1)
    wait_gather(q)
    wait_store(blks[q], q)
    transpose(rows[q], blks[q])
    start_store(c, blks[q], q)
  for q in range(2):
    wait_store(blks[q], q)


def kernel(indices, table):
  batch, seq = indices.shape
  n = batch * seq
  vocab = table.shape[0]
  v8 = (vocab + 7) // 8 * 8

  idx_t = (indices.T.reshape(n) * 2).astype(jnp.int32)
  tabv = jnp.pad(table, ((0, v8 - vocab), (0, 2 * DIM - table.shape[1])))
  tabv = tabv.reshape(2 * v8, DIM)

  mesh = plsc.VectorSubcoreMesh(core_axis_name="c", subcore_axis_name="s")
  k = functools.partial(
      pl.kernel,
      mesh=mesh,
      out_type=jax.ShapeDtypeStruct((n * DIM,), jnp.float32),
      scratch_types=[
          pltpu.VMEM((BLOCKS_PER_W * BLK,), jnp.int32),
          pltpu.VMEM((CHUNK, DIM), jnp.float32),
          pltpu.VMEM((CHUNK, DIM), jnp.float32),
          pltpu.VMEM((2 * DIM * BLK,), jnp.float32),
          pltpu.VMEM((2 * DIM * BLK,), jnp.float32),
          pltpu.SemaphoreType.DMA((2,)),
          pltpu.SemaphoreType.DMA((2,)),
      ],
      compiler_params=pltpu.CompilerParams(
          use_tc_tiling_on_sc=False, needs_layout_passes=False),
  )(_body)

  out_flat = k(idx_t, tabv)
  out5 = out_flat.reshape(seq, DIM // 8, batch // BLK, 8, BLK)
  return out5.transpose(2, 4, 0, 1, 3).reshape(batch, seq, DIM)
